# baseline (speedup 1.0000x reference)
import functools
import jax
import jax.numpy as jnp
from jax.experimental import pallas as pl
from jax.experimental.pallas import tpu as pltpu

H = 8
D = 16
HID = 128
OUT = 64
N = 50000
BN = 1000


def _mk_m16(att):
    m = jnp.zeros((HID, 16), jnp.float32)
    return m.at[jnp.arange(HID), jnp.arange(HID) // D].set(att.reshape(HID))


def _r_expand():
    return jnp.zeros((8, HID), jnp.float32).at[
        jnp.arange(HID) // D, jnp.arange(HID)
    ].set(1.0)


def _proj1_body(x_ref, w_ref, b_ref, ms_ref, md_ref, xn_ref, as_ref, ad_ref):
    xn = jnp.dot(x_ref[...], w_ref[...], preferred_element_type=jnp.float32)
    xn = xn + b_ref[...]
    xn_ref[...] = xn
    as_ref[...] = jnp.dot(xn, ms_ref[...], preferred_element_type=jnp.float32)
    ad_ref[...] = jnp.dot(xn, md_ref[...], preferred_element_type=jnp.float32)


def _proj1(x, w, b, ms, md):
    return pl.pallas_call(
        _proj1_body,
        grid=(N // BN,),
        in_specs=[
            pl.BlockSpec((BN, HID), lambda i: (i, 0)),
            pl.BlockSpec((HID, HID), lambda i: (0, 0)),
            pl.BlockSpec((1, HID), lambda i: (0, 0)),
            pl.BlockSpec((HID, 16), lambda i: (0, 0)),
            pl.BlockSpec((HID, 16), lambda i: (0, 0)),
        ],
        out_specs=[
            pl.BlockSpec((BN, HID), lambda i: (i, 0)),
            pl.BlockSpec((BN, 16), lambda i: (i, 0)),
            pl.BlockSpec((BN, 16), lambda i: (i, 0)),
        ],
        out_shape=[
            jax.ShapeDtypeStruct((N, HID), jnp.float32),
            jax.ShapeDtypeStruct((N, 16), jnp.float32),
            jax.ShapeDtypeStruct((N, 16), jnp.float32),
        ],
    )(x, w, b.reshape(1, HID), ms, md)


def _proj2_body(u_ref, s_ref, r_ref, w_ref, b_ref, ms_ref, md_ref,
                xn_ref, as_ref, ad_ref):
    srep = jnp.dot(s_ref[...], r_ref[...], preferred_element_type=jnp.float32)
    xi = jnp.maximum(u_ref[...] / (srep + 1e-16), 0.0)
    xn = jnp.dot(xi, w_ref[...], preferred_element_type=jnp.float32)
    xn = xn + b_ref[...]
    xn_ref[...] = xn
    as_ref[...] = jnp.dot(xn, ms_ref[...], preferred_element_type=jnp.float32)
    ad_ref[...] = jnp.dot(xn, md_ref[...], preferred_element_type=jnp.float32)


def _proj2(u, s8, w, b, ms, md):
    return pl.pallas_call(
        _proj2_body,
        grid=(N // BN,),
        in_specs=[
            pl.BlockSpec((BN, HID), lambda i: (i, 0)),
            pl.BlockSpec((BN, 8), lambda i: (i, 0)),
            pl.BlockSpec((8, HID), lambda i: (0, 0)),
            pl.BlockSpec((HID, HID), lambda i: (0, 0)),
            pl.BlockSpec((1, HID), lambda i: (0, 0)),
            pl.BlockSpec((HID, 16), lambda i: (0, 0)),
            pl.BlockSpec((HID, 16), lambda i: (0, 0)),
        ],
        out_specs=[
            pl.BlockSpec((BN, HID), lambda i: (i, 0)),
            pl.BlockSpec((BN, 16), lambda i: (i, 0)),
            pl.BlockSpec((BN, 16), lambda i: (i, 0)),
        ],
        out_shape=[
            jax.ShapeDtypeStruct((N, HID), jnp.float32),
            jax.ShapeDtypeStruct((N, 16), jnp.float32),
            jax.ShapeDtypeStruct((N, 16), jnp.float32),
        ],
    )(u, s8, _r_expand(), w, b.reshape(1, HID), ms, md)


def _final_body(u_ref, s_ref, r_ref, w_ref, b_ref, o_ref):
    srep = jnp.dot(s_ref[...], r_ref[...], preferred_element_type=jnp.float32)
    xi = jnp.maximum(u_ref[...] / (srep + 1e-16), 0.0)
    o_ref[...] = jnp.dot(xi, w_ref[...],
                         preferred_element_type=jnp.float32) + b_ref[...]


def _final(u, s8, w, b):
    return pl.pallas_call(
        _final_body,
        grid=(N // BN,),
        in_specs=[
            pl.BlockSpec((BN, HID), lambda i: (i, 0)),
            pl.BlockSpec((BN, 8), lambda i: (i, 0)),
            pl.BlockSpec((8, HID), lambda i: (0, 0)),
            pl.BlockSpec((HID, OUT), lambda i: (0, 0)),
            pl.BlockSpec((1, OUT), lambda i: (0, 0)),
        ],
        out_specs=pl.BlockSpec((BN, OUT), lambda i: (i, 0)),
        out_shape=jax.ShapeDtypeStruct((N, OUT), jnp.float32),
    )(u, s8, _r_expand(), w, b.reshape(1, OUT))


def _edges_jnp(as16, ad16, xn, src, dst):
    al = as16[src, :8] + ad16[dst, :8]
    al = jnp.where(al >= 0, al, 0.2 * al)
    ex = jnp.exp(al)
    s = jax.ops.segment_sum(ex, dst, num_segments=N)
    msg = xn[src].reshape(-1, H, D) * ex[:, :, None]
    unnorm = jax.ops.segment_sum(msg.reshape(-1, HID), dst, num_segments=N)
    return unnorm, s


def kernel(x_author, x_paper, edge_index_writes, edge_index_rev_writes, params):
    p1, p2 = params["layers"]
    kw = "author__writes__paper"
    kr = "paper__rev_writes__author"

    sw, dw = edge_index_writes[0], edge_index_writes[1]
    sr, dr = edge_index_rev_writes[0], edge_index_rev_writes[1]

    def layer(xa_u, xa_s, xp_u, xp_s, p, first):
        ms_a = _mk_m16(p["att_src"][kw])
        md_a = _mk_m16(p["att_dst"][kr])
        ms_p = _mk_m16(p["att_src"][kr])
        md_p = _mk_m16(p["att_dst"][kw])
        if first:
            xn_a, as_a, ad_a = _proj1(xa_u, p["proj"]["author"]["W"],
                                      p["proj"]["author"]["b"], ms_a, md_a)
            xn_p, as_p, ad_p = _proj1(xp_u, p["proj"]["paper"]["W"],
                                      p["proj"]["paper"]["b"], ms_p, md_p)
        else:
            xn_a, as_a, ad_a = _proj2(xa_u, xa_s, p["proj"]["author"]["W"],
                                      p["proj"]["author"]["b"], ms_a, md_a)
            xn_p, as_p, ad_p = _proj2(xp_u, xp_s, p["proj"]["paper"]["W"],
                                      p["proj"]["paper"]["b"], ms_p, md_p)
        u_p, s_p = _edges_jnp(as_a, ad_p, xn_a, sw, dw)
        u_a, s_a = _edges_jnp(as_p, ad_a, xn_p, sr, dr)
        return u_a, s_a, u_p, s_p

    u_a, s_a, u_p, s_p = layer(x_author, None, x_paper, None, p1, True)
    u_a2, s_a2, u_p2, s_p2 = layer(u_a, s_a, u_p, s_p, p2, False)
    return _final(u_p2, s_p2, params["lin_W"], params["lin_b"])


# speedup vs baseline: 52.6495x; 52.6495x over previous
import functools
import jax
import jax.numpy as jnp
from jax import lax
from jax.experimental import pallas as pl
from jax.experimental.pallas import tpu as pltpu
from jax.experimental.pallas import tpu_sc as plsc

H = 8
D = 16
HID = 128
OUT = 64
N = 50000
BN = 1000
BN2 = 400
NPAD = 51200
RS = NPAD // 16
E = 300000
W = 512
NWIN = 19
EPAD = 32 * NWIN * W
EROWS = EPAD // 128


def _mk_m16(att):
    m = jnp.zeros((HID, 16), jnp.float32)
    return m.at[jnp.arange(HID), jnp.arange(HID) // D].set(att.reshape(HID))


def _r_expand():
    return jnp.zeros((16, HID), jnp.float32).at[
        jnp.arange(HID) // D, jnp.arange(HID)
    ].set(1.0)


def _p_place():
    p = jnp.zeros((8, 16, HID), jnp.float32)
    return p.at[jnp.arange(HID) // D, jnp.arange(HID) % D,
                jnp.arange(HID)].set(1.0)


def _proj1_body(x_ref, w_ref, b_ref, ms_ref, md_ref, xn_ref, as_ref, ad_ref):
    xn = jnp.dot(x_ref[...], w_ref[...], preferred_element_type=jnp.float32)
    xn = xn + b_ref[...]
    xn_ref[...] = xn
    as_ref[...] = jnp.dot(xn, ms_ref[...], preferred_element_type=jnp.float32)
    ad_ref[...] = jnp.dot(xn, md_ref[...], preferred_element_type=jnp.float32)


def _proj1(x, w, b, ms, md):
    return pl.pallas_call(
        _proj1_body,
        grid=(N // BN,),
        in_specs=[
            pl.BlockSpec((BN, HID), lambda i: (i, 0)),
            pl.BlockSpec((HID, HID), lambda i: (0, 0)),
            pl.BlockSpec((1, HID), lambda i: (0, 0)),
            pl.BlockSpec((HID, 16), lambda i: (0, 0)),
            pl.BlockSpec((HID, 16), lambda i: (0, 0)),
        ],
        out_specs=[
            pl.BlockSpec((BN, HID), lambda i: (i, 0)),
            pl.BlockSpec((BN, 16), lambda i: (i, 0)),
            pl.BlockSpec((BN, 16), lambda i: (i, 0)),
        ],
        out_shape=[
            jax.ShapeDtypeStruct((N, HID), jnp.float32),
            jax.ShapeDtypeStruct((N, 16), jnp.float32),
            jax.ShapeDtypeStruct((N, 16), jnp.float32),
        ],
    )(x, w, b.reshape(1, HID), ms, md)


def _combine(u_ref, s_ref, p_ref, r_ref):
    u = jnp.dot(u_ref[0, 0] + u_ref[1, 0], p_ref[0],
                preferred_element_type=jnp.float32)
    for h in range(1, 8):
        u = u + jnp.dot(u_ref[0, h] + u_ref[1, h], p_ref[h],
                        preferred_element_type=jnp.float32)
    srep = jnp.dot(s_ref[0] + s_ref[1], r_ref[...],
                   preferred_element_type=jnp.float32)
    return jnp.maximum(u / (srep + 1e-16), 0.0)


def _proj2_body(u_ref, s_ref, p_ref, r_ref, w_ref, b_ref, ms_ref, md_ref,
                xn_ref, as_ref, ad_ref):
    xi = _combine(u_ref, s_ref, p_ref, r_ref)
    xn = jnp.dot(xi, w_ref[...], preferred_element_type=jnp.float32)
    xn = xn + b_ref[...]
    xn_ref[...] = xn
    as_ref[...] = jnp.dot(xn, ms_ref[...], preferred_element_type=jnp.float32)
    ad_ref[...] = jnp.dot(xn, md_ref[...], preferred_element_type=jnp.float32)


_COMBINE_SPECS = [
    pl.BlockSpec((2, 8, BN2, 16), lambda i: (0, 0, i, 0)),
    pl.BlockSpec((2, BN2, 16), lambda i: (0, i, 0)),
    pl.BlockSpec((8, 16, HID), lambda i: (0, 0, 0)),
    pl.BlockSpec((16, HID), lambda i: (0, 0)),
]


def _proj2(u4, s32, w, b, ms, md):
    return pl.pallas_call(
        _proj2_body,
        grid=(N // BN2,),
        in_specs=_COMBINE_SPECS + [
            pl.BlockSpec((HID, HID), lambda i: (0, 0)),
            pl.BlockSpec((1, HID), lambda i: (0, 0)),
            pl.BlockSpec((HID, 16), lambda i: (0, 0)),
            pl.BlockSpec((HID, 16), lambda i: (0, 0)),
        ],
        out_specs=[
            pl.BlockSpec((BN2, HID), lambda i: (i, 0)),
            pl.BlockSpec((BN2, 16), lambda i: (i, 0)),
            pl.BlockSpec((BN2, 16), lambda i: (i, 0)),
        ],
        out_shape=[
            jax.ShapeDtypeStruct((N, HID), jnp.float32),
            jax.ShapeDtypeStruct((N, 16), jnp.float32),
            jax.ShapeDtypeStruct((N, 16), jnp.float32),
        ],
    )(u4, s32, _p_place(), _r_expand(), w, b.reshape(1, HID), ms, md)


def _final_body(u_ref, s_ref, p_ref, r_ref, w_ref, b_ref, o_ref):
    xi = _combine(u_ref, s_ref, p_ref, r_ref)
    o_ref[...] = jnp.dot(xi, w_ref[...],
                         preferred_element_type=jnp.float32) + b_ref[...]


def _final(u4, s32, w, b):
    return pl.pallas_call(
        _final_body,
        grid=(N // BN2,),
        in_specs=_COMBINE_SPECS + [
            pl.BlockSpec((HID, OUT), lambda i: (0, 0)),
            pl.BlockSpec((1, OUT), lambda i: (0, 0)),
        ],
        out_specs=pl.BlockSpec((BN2, OUT), lambda i: (i, 0)),
        out_shape=jax.ShapeDtypeStruct((N, OUT), jnp.float32),
    )(u4, s32, _p_place(), _r_expand(), w, b.reshape(1, OUT))


def _edges_sc(as0, ad0, xs0, sr0, ds0, as1, ad1, xs1, sr1, ds1):
    mesh = plsc.VectorSubcoreMesh(core_axis_name="c", subcore_axis_name="s")

    @functools.partial(
        pl.kernel,
        out_type=[
            jax.ShapeDtypeStruct((2, 2, NPAD, 16), jnp.float32),
            jax.ShapeDtypeStruct((2, 2, 8, NPAD, 16), jnp.float32),
            jax.ShapeDtypeStruct((2, EROWS, 128, 16), jnp.float32),
        ],
        mesh=mesh,
        scratch_types=[
            pltpu.VMEM((4, 128), jnp.int32),
            pltpu.VMEM((4, 128), jnp.int32),
            pltpu.VMEM((4, 128), jnp.int32),
            pltpu.VMEM((4, 128, 16), jnp.float32),
            pltpu.VMEM((4, 128, 16), jnp.float32),
            pltpu.VMEM((4, 128, 16), jnp.float32),
            pltpu.VMEM((4, 128, 16), jnp.float32),
            pltpu.VMEM((4, 128, 16), jnp.float32),
            pltpu.VMEM((640, 16), jnp.float32),
            pltpu.VMEM_SHARED((NPAD, 16), jnp.float32),
            pltpu.SemaphoreType.DMA,
            pltpu.SemaphoreType.DMA,
        ],
        compiler_params=pltpu.CompilerParams(use_tc_tiling_on_sc=False),
    )
    def ek(as0r, ad0r, xs0r, sr0r, ds0r, as1r, ad1r, xs1r, sr1r, ds1r,
           s_out, un_out, ex_out,
           idx_s, idx_d, idx8, arows, brows, exw, exr, xrows, zz, agg,
           sem, sem2):
        cid = lax.axis_index("c")
        sid = lax.axis_index("s")
        wid = cid * 16 + sid
        rbase = sid * RS
        zvec = jnp.zeros((16,), jnp.float32)

        @pl.loop(0, 640)
        def _(i):
            zz[i, pl.ds(0, 16)] = zvec

        def zero_slice():
            for j in range(5):
                pltpu.sync_copy(zz, agg.at[pl.ds(rbase + j * 640, 640)])

        for et, (asr, adr, xsr, srr, dsr) in enumerate(
            ((as0r, ad0r, xs0r, sr0r, ds0r),
             (as1r, ad1r, xs1r, sr1r, ds1r))):
            zero_slice()
            plsc.subcore_barrier()

            @pl.loop(0, NWIN)
            def _(w):
                bw = (wid * NWIN + w) * 4
                pltpu.sync_copy(srr.at[pl.ds(bw, 4)], idx_s)
                pltpu.sync_copy(dsr.at[pl.ds(bw, 4)], idx_d)
                cps = [pltpu.async_copy(asr.at[idx_s.at[k]], arows.at[k], sem)
                       for k in range(4)]
                cps += [pltpu.async_copy(adr.at[idx_d.at[k]], brows.at[k],
                                         sem2) for k in range(4)]
                for cp in cps:
                    cp.wait()

                @pl.loop(0, 128)
                def _(r):
                    for k in range(4):
                        t0 = arows[k, r, :] + brows[k, r, :]
                        t0 = jnp.where(t0 >= 0.0, t0, t0 * 0.2)
                        exw[k, r, :] = jnp.exp(t0)

                for k in range(4):
                    pltpu.sync_copy(exw.at[k], agg.at[idx_d.at[k]], add=True)
                pltpu.sync_copy(exw, ex_out.at[et, pl.ds(bw, 4)])

            plsc.subcore_barrier()
            pltpu.sync_copy(agg.at[pl.ds(rbase, RS)],
                            s_out.at[et, cid, pl.ds(rbase, RS)])
            zero_slice()
            plsc.subcore_barrier()

            for h in range(8):
                @pl.loop(0, NWIN)
                def _(w):
                    bw = (wid * NWIN + w) * 4
                    pltpu.sync_copy(srr.at[pl.ds(bw, 4)], idx_s)
                    pltpu.sync_copy(dsr.at[pl.ds(bw, 4)], idx_d)
                    for k in range(4):
                        for j in range(8):
                            sl = pl.ds(j * 16, 16)
                            idx8[k, sl] = idx_s[k, sl] * 8 + h
                    cps = [pltpu.async_copy(xsr.at[idx8.at[k]], xrows.at[k],
                                            sem) for k in range(4)]
                    pltpu.sync_copy(ex_out.at[et, pl.ds(bw, 4)], exr)
                    for cp in cps:
                        cp.wait()

                    @pl.loop(0, 128)
                    def _(r):
                        for k in range(4):
                            ev = exr[k, r, :]
                            xrows[k, r, :] = xrows[k, r, :] * ev[h]

                    for k in range(4):
                        pltpu.sync_copy(xrows.at[k], agg.at[idx_d.at[k]],
                                        add=True)

                plsc.subcore_barrier()
                pltpu.sync_copy(agg.at[pl.ds(rbase, RS)],
                                un_out.at[et, cid, h, pl.ds(rbase, RS)])
                zero_slice()
                plsc.subcore_barrier()

    return ek(as0, ad0, xs0, sr0, ds0, as1, ad1, xs1, sr1, ds1)


def _pad_idx(a, off):
    pad = jnp.arange(EPAD - E, dtype=jnp.int32)
    fill = off + (pad % 1024)
    return jnp.concatenate([a.astype(jnp.int32), fill]).reshape(EROWS, 128)


def kernel(x_author, x_paper, edge_index_writes, edge_index_rev_writes, params):
    p1, p2 = params["layers"]
    kw = "author__writes__paper"
    kr = "paper__rev_writes__author"

    sw = _pad_idx(edge_index_writes[0], 0)
    dw = _pad_idx(edge_index_writes[1], N)
    sr = _pad_idx(edge_index_rev_writes[0], 0)
    dr = _pad_idx(edge_index_rev_writes[1], N)

    def layer(proj_a, proj_p, p):
        xn_a, as_a, ad_a = proj_a
        xn_p, as_p, ad_p = proj_p
        s_out, un_out, _ = _edges_sc(
            as_a, ad_p, xn_a.reshape(8 * N, 16), sw, dw,
            as_p, ad_a, xn_p.reshape(8 * N, 16), sr, dr,
        )
        return (un_out[1], s_out[1]), (un_out[0], s_out[0])

    def mk_ms(p):
        return (_mk_m16(p["att_src"][kw]), _mk_m16(p["att_dst"][kr]),
                _mk_m16(p["att_src"][kr]), _mk_m16(p["att_dst"][kw]))

    ms_a, md_a, ms_p, md_p = mk_ms(p1)
    proj_a = _proj1(x_author, p1["proj"]["author"]["W"],
                    p1["proj"]["author"]["b"], ms_a, md_a)
    proj_p = _proj1(x_paper, p1["proj"]["paper"]["W"],
                    p1["proj"]["paper"]["b"], ms_p, md_p)
    (u_a, s_a), (u_p, s_p) = layer(proj_a, proj_p, p1)

    ms_a, md_a, ms_p, md_p = mk_ms(p2)
    proj_a2 = _proj2(u_a, s_a, p2["proj"]["author"]["W"],
                     p2["proj"]["author"]["b"], ms_a, md_a)
    proj_p2 = _proj2(u_p, s_p, p2["proj"]["paper"]["W"],
                     p2["proj"]["paper"]["b"], ms_p, md_p)
    (u_a2, s_a2), (u_p2, s_p2) = layer(proj_a2, proj_p2, p2)

    return _final(u_p2, s_p2, params["lin_W"], params["lin_b"])


# speedup vs baseline: 71.4910x; 1.3579x over previous
import functools
import jax
import jax.numpy as jnp
from jax import lax
from jax.experimental import pallas as pl
from jax.experimental.pallas import tpu as pltpu
from jax.experimental.pallas import tpu_sc as plsc

H = 8
D = 16
HID = 128
OUT = 64
N = 50000
BN = 1000
BN2 = 400
NPAD = 51200
RS = NPAD // 16
E = 300000
W = 512
NWIN = 19
EPAD = 32 * NWIN * W
EROWS = EPAD // 128


def _mk_m16(att):
    m = jnp.zeros((HID, 16), jnp.float32)
    return m.at[jnp.arange(HID), jnp.arange(HID) // D].set(att.reshape(HID))


def _r_expand():
    return jnp.zeros((16, HID), jnp.float32).at[
        jnp.arange(HID) // D, jnp.arange(HID)
    ].set(1.0)


def _p_place():
    p = jnp.zeros((8, 16, HID), jnp.float32)
    return p.at[jnp.arange(HID) // D, jnp.arange(HID) % D,
                jnp.arange(HID)].set(1.0)


def _proj1_body(x_ref, w_ref, b_ref, ms_ref, md_ref, xn_ref, as_ref, ad_ref):
    xn = jnp.dot(x_ref[...], w_ref[...], preferred_element_type=jnp.float32)
    xn = xn + b_ref[...]
    xn_ref[...] = xn
    as_ref[...] = jnp.dot(xn, ms_ref[...], preferred_element_type=jnp.float32)
    ad_ref[...] = jnp.dot(xn, md_ref[...], preferred_element_type=jnp.float32)


def _proj1(x, w, b, ms, md):
    return pl.pallas_call(
        _proj1_body,
        grid=(N // BN,),
        in_specs=[
            pl.BlockSpec((BN, HID), lambda i: (i, 0)),
            pl.BlockSpec((HID, HID), lambda i: (0, 0)),
            pl.BlockSpec((1, HID), lambda i: (0, 0)),
            pl.BlockSpec((HID, 16), lambda i: (0, 0)),
            pl.BlockSpec((HID, 16), lambda i: (0, 0)),
        ],
        out_specs=[
            pl.BlockSpec((BN, HID), lambda i: (i, 0)),
            pl.BlockSpec((BN, 16), lambda i: (i, 0)),
            pl.BlockSpec((BN, 16), lambda i: (i, 0)),
        ],
        out_shape=[
            jax.ShapeDtypeStruct((N, HID), jnp.float32),
            jax.ShapeDtypeStruct((N, 16), jnp.float32),
            jax.ShapeDtypeStruct((N, 16), jnp.float32),
        ],
    )(x, w, b.reshape(1, HID), ms, md)


def _combine(u_ref, s_ref, p_ref, r_ref):
    u = jnp.dot(u_ref[0, 0] + u_ref[1, 0], p_ref[0],
                preferred_element_type=jnp.float32)
    for h in range(1, 8):
        u = u + jnp.dot(u_ref[0, h] + u_ref[1, h], p_ref[h],
                        preferred_element_type=jnp.float32)
    srep = jnp.dot(s_ref[0] + s_ref[1], r_ref[...],
                   preferred_element_type=jnp.float32)
    return jnp.maximum(u / (srep + 1e-16), 0.0)


def _proj2_body(u_ref, s_ref, p_ref, r_ref, w_ref, b_ref, ms_ref, md_ref,
                xn_ref, as_ref, ad_ref):
    xi = _combine(u_ref, s_ref, p_ref, r_ref)
    xn = jnp.dot(xi, w_ref[...], preferred_element_type=jnp.float32)
    xn = xn + b_ref[...]
    xn_ref[...] = xn
    as_ref[...] = jnp.dot(xn, ms_ref[...], preferred_element_type=jnp.float32)
    ad_ref[...] = jnp.dot(xn, md_ref[...], preferred_element_type=jnp.float32)


_COMBINE_SPECS = [
    pl.BlockSpec((2, 8, BN2, 16), lambda i: (0, 0, i, 0)),
    pl.BlockSpec((2, BN2, 16), lambda i: (0, i, 0)),
    pl.BlockSpec((8, 16, HID), lambda i: (0, 0, 0)),
    pl.BlockSpec((16, HID), lambda i: (0, 0)),
]


def _proj2(u4, s32, w, b, ms, md):
    return pl.pallas_call(
        _proj2_body,
        grid=(N // BN2,),
        in_specs=_COMBINE_SPECS + [
            pl.BlockSpec((HID, HID), lambda i: (0, 0)),
            pl.BlockSpec((1, HID), lambda i: (0, 0)),
            pl.BlockSpec((HID, 16), lambda i: (0, 0)),
            pl.BlockSpec((HID, 16), lambda i: (0, 0)),
        ],
        out_specs=[
            pl.BlockSpec((BN2, HID), lambda i: (i, 0)),
            pl.BlockSpec((BN2, 16), lambda i: (i, 0)),
            pl.BlockSpec((BN2, 16), lambda i: (i, 0)),
        ],
        out_shape=[
            jax.ShapeDtypeStruct((N, HID), jnp.float32),
            jax.ShapeDtypeStruct((N, 16), jnp.float32),
            jax.ShapeDtypeStruct((N, 16), jnp.float32),
        ],
    )(u4, s32, _p_place(), _r_expand(), w, b.reshape(1, HID), ms, md)


def _final_body(u_ref, s_ref, p_ref, r_ref, w_ref, b_ref, o_ref):
    xi = _combine(u_ref, s_ref, p_ref, r_ref)
    o_ref[...] = jnp.dot(xi, w_ref[...],
                         preferred_element_type=jnp.float32) + b_ref[...]


def _final(u4, s32, w, b):
    return pl.pallas_call(
        _final_body,
        grid=(N // BN2,),
        in_specs=_COMBINE_SPECS + [
            pl.BlockSpec((HID, OUT), lambda i: (0, 0)),
            pl.BlockSpec((1, OUT), lambda i: (0, 0)),
        ],
        out_specs=pl.BlockSpec((BN2, OUT), lambda i: (i, 0)),
        out_shape=jax.ShapeDtypeStruct((N, OUT), jnp.float32),
    )(u4, s32, _p_place(), _r_expand(), w, b.reshape(1, OUT))


def _edges_sc(as0, ad0, xs0, sr0, ds0, as1, ad1, xs1, sr1, ds1):
    mesh = plsc.VectorSubcoreMesh(core_axis_name="c", subcore_axis_name="s")

    @functools.partial(
        pl.kernel,
        out_type=[
            jax.ShapeDtypeStruct((2, 2, NPAD, 16), jnp.float32),
            jax.ShapeDtypeStruct((2, 2, 8, NPAD, 16), jnp.float32),
            jax.ShapeDtypeStruct((2, EROWS, 128, 16), jnp.float32),
        ],
        mesh=mesh,
        scratch_types=[
            pltpu.VMEM((4, 128), jnp.int32),
            pltpu.VMEM((4, 128, 16), jnp.float32),
            pltpu.VMEM((4, 128, 16), jnp.float32),
            pltpu.VMEM((4, 128, 16), jnp.float32),
            pltpu.VMEM((160, 16), jnp.float32),
            pltpu.VMEM_SHARED((NPAD, 16), jnp.float32),
            pltpu.SemaphoreType.DMA,
            pltpu.SemaphoreType.DMA,
        ],
        compiler_params=pltpu.CompilerParams(use_tc_tiling_on_sc=False),
    )
    def ek(as0r, ad0r, xs0r, sr0r, ds0r, as1r, ad1r, xs1r, sr1r, ds1r,
           s_out, un_out, ex_out,
           idx8, arows, brows, xrows, zz, agg,
           sem, sem2):
        cid = lax.axis_index("c")
        sid = lax.axis_index("s")
        rbase = sid * RS
        zvec = jnp.zeros((16,), jnp.float32)
        GRID = EPAD // W

        @pl.loop(0, 160)
        def _(i):
            zz[i, :] = zvec

        def zero_slice():
            for j in range(20):
                pltpu.sync_copy(zz, agg.at[pl.ds(rbase + j * 160, 160)])

        idx_specs = [pl.BlockSpec((4, 128), lambda w: (w, 0)),
                     pl.BlockSpec((4, 128), lambda w: (w, 0))]
        ex_spec = pl.BlockSpec((4, 128, 16), lambda w: (w, 0, 0))

        for et, (asr, adr, xsr, srr, dsr) in enumerate(
            ((as0r, ad0r, xs0r, sr0r, ds0r),
             (as1r, ad1r, xs1r, sr1r, ds1r))):
            zero_slice()
            plsc.subcore_barrier()

            def p1_body(idx_s_v, idx_d_v, ex_v, asr=asr, adr=adr):
                cps = [pltpu.async_copy(asr.at[idx_s_v.at[k]], arows.at[k],
                                        sem) for k in range(4)]
                cps += [pltpu.async_copy(adr.at[idx_d_v.at[k]], brows.at[k],
                                         sem2) for k in range(4)]
                for cp in cps:
                    cp.wait()

                @pl.loop(0, 128)
                def _(r):
                    for k in range(4):
                        t0 = arows[k, r, :] + brows[k, r, :]
                        t0 = jnp.where(t0 >= 0.0, t0, t0 * 0.2)
                        ex_v[k, r, :] = jnp.exp(t0)

                for k in range(4):
                    pltpu.sync_copy(ex_v.at[k], agg.at[idx_d_v.at[k]],
                                    add=True)

            pltpu.emit_pipeline(
                p1_body,
                grid=(GRID,),
                in_specs=idx_specs,
                out_specs=[ex_spec],
                core_axis_name=("c", "s"),
                dimension_semantics=(pltpu.PARALLEL,),
            )(srr, dsr, ex_out.at[et])

            plsc.subcore_barrier()
            pltpu.sync_copy(agg.at[pl.ds(rbase, RS)],
                            s_out.at[et, cid, pl.ds(rbase, RS)])
            zero_slice()
            plsc.subcore_barrier()

            for h in range(8):
                def p2_body(idx_s_v, idx_d_v, ex_v, h=h, xsr=xsr):
                    for k in range(4):
                        for j in range(8):
                            sl = pl.ds(j * 16, 16)
                            idx8[k, sl] = idx_s_v[k, sl] * 8 + h
                    cps = [pltpu.async_copy(xsr.at[idx8.at[k]], xrows.at[k],
                                            sem) for k in range(4)]
                    for cp in cps:
                        cp.wait()

                    @pl.loop(0, 128)
                    def _(r):
                        for k in range(4):
                            ev = ex_v[k, r, :]
                            xrows[k, r, :] = xrows[k, r, :] * ev[h]

                    for k in range(4):
                        pltpu.sync_copy(xrows.at[k], agg.at[idx_d_v.at[k]],
                                        add=True)

                pltpu.emit_pipeline(
                    p2_body,
                    grid=(GRID,),
                    in_specs=idx_specs + [ex_spec],
                    out_specs=[],
                    core_axis_name=("c", "s"),
                    dimension_semantics=(pltpu.PARALLEL,),
                )(srr, dsr, ex_out.at[et])

                plsc.subcore_barrier()
                pltpu.sync_copy(agg.at[pl.ds(rbase, RS)],
                                un_out.at[et, cid, h, pl.ds(rbase, RS)])
                zero_slice()
                plsc.subcore_barrier()

    return ek(as0, ad0, xs0, sr0, ds0, as1, ad1, xs1, sr1, ds1)


def _pad_idx(a, off):
    pad = jnp.arange(EPAD - E, dtype=jnp.int32)
    fill = off + (pad % 1024)
    return jnp.concatenate([a.astype(jnp.int32), fill]).reshape(EROWS, 128)


def kernel(x_author, x_paper, edge_index_writes, edge_index_rev_writes, params):
    p1, p2 = params["layers"]
    kw = "author__writes__paper"
    kr = "paper__rev_writes__author"

    sw = _pad_idx(edge_index_writes[0], 0)
    dw = _pad_idx(edge_index_writes[1], N)
    sr = _pad_idx(edge_index_rev_writes[0], 0)
    dr = _pad_idx(edge_index_rev_writes[1], N)

    def layer(proj_a, proj_p, p):
        xn_a, as_a, ad_a = proj_a
        xn_p, as_p, ad_p = proj_p
        s_out, un_out, _ = _edges_sc(
            as_a, ad_p, xn_a.reshape(8 * N, 16), sw, dw,
            as_p, ad_a, xn_p.reshape(8 * N, 16), sr, dr,
        )
        return (un_out[1], s_out[1]), (un_out[0], s_out[0])

    def mk_ms(p):
        return (_mk_m16(p["att_src"][kw]), _mk_m16(p["att_dst"][kr]),
                _mk_m16(p["att_src"][kr]), _mk_m16(p["att_dst"][kw]))

    ms_a, md_a, ms_p, md_p = mk_ms(p1)
    proj_a = _proj1(x_author, p1["proj"]["author"]["W"],
                    p1["proj"]["author"]["b"], ms_a, md_a)
    proj_p = _proj1(x_paper, p1["proj"]["paper"]["W"],
                    p1["proj"]["paper"]["b"], ms_p, md_p)
    (u_a, s_a), (u_p, s_p) = layer(proj_a, proj_p, p1)

    ms_a, md_a, ms_p, md_p = mk_ms(p2)
    proj_a2 = _proj2(u_a, s_a, p2["proj"]["author"]["W"],
                     p2["proj"]["author"]["b"], ms_a, md_a)
    proj_p2 = _proj2(u_p, s_p, p2["proj"]["paper"]["W"],
                     p2["proj"]["paper"]["b"], ms_p, md_p)
    (u_a2, s_a2), (u_p2, s_p2) = layer(proj_a2, proj_p2, p2)

    return _final(u_p2, s_p2, params["lin_W"], params["lin_b"])


# speedup vs baseline: 75.7931x; 1.0602x over previous
import functools
import jax
import jax.numpy as jnp
from jax import lax
from jax.experimental import pallas as pl
from jax.experimental.pallas import tpu as pltpu
from jax.experimental.pallas import tpu_sc as plsc

H = 8
D = 16
HID = 128
OUT = 64
N = 50000
BN = 1000
BN2 = 400
NPAD = 51200
RS = NPAD // 16
E = 300000
W = 512
NWIN = 19
EPAD = 32 * NWIN * W
EROWS = EPAD // 128


def _mk_m16(att):
    m = jnp.zeros((HID, 16), jnp.float32)
    return m.at[jnp.arange(HID), jnp.arange(HID) // D].set(att.reshape(HID))


def _r_expand():
    return jnp.zeros((16, HID), jnp.float32).at[
        jnp.arange(HID) // D, jnp.arange(HID)
    ].set(1.0)


def _p_place():
    p = jnp.zeros((8, 16, HID), jnp.float32)
    return p.at[jnp.arange(HID) // D, jnp.arange(HID) % D,
                jnp.arange(HID)].set(1.0)


def _proj1_body(x_ref, w_ref, b_ref, ms_ref, md_ref, xn_ref, as_ref, ad_ref):
    xn = jnp.dot(x_ref[...], w_ref[...], preferred_element_type=jnp.float32)
    xn = xn + b_ref[...]
    xn_ref[...] = xn
    as_ref[...] = jnp.dot(xn, ms_ref[...], preferred_element_type=jnp.float32)
    ad_ref[...] = jnp.dot(xn, md_ref[...], preferred_element_type=jnp.float32)


def _proj1(x, w, b, ms, md):
    return pl.pallas_call(
        _proj1_body,
        grid=(N // BN,),
        in_specs=[
            pl.BlockSpec((BN, HID), lambda i: (i, 0)),
            pl.BlockSpec((HID, HID), lambda i: (0, 0)),
            pl.BlockSpec((1, HID), lambda i: (0, 0)),
            pl.BlockSpec((HID, 16), lambda i: (0, 0)),
            pl.BlockSpec((HID, 16), lambda i: (0, 0)),
        ],
        out_specs=[
            pl.BlockSpec((BN, HID), lambda i: (i, 0)),
            pl.BlockSpec((BN, 16), lambda i: (i, 0)),
            pl.BlockSpec((BN, 16), lambda i: (i, 0)),
        ],
        out_shape=[
            jax.ShapeDtypeStruct((N, HID), jnp.float32),
            jax.ShapeDtypeStruct((N, 16), jnp.float32),
            jax.ShapeDtypeStruct((N, 16), jnp.float32),
        ],
    )(x, w, b.reshape(1, HID), ms, md)


def _combine(u_ref, s_ref, p_ref, r_ref):
    u = jnp.dot(u_ref[0, 0] + u_ref[1, 0], p_ref[0],
                preferred_element_type=jnp.float32)
    for h in range(1, 8):
        u = u + jnp.dot(u_ref[0, h] + u_ref[1, h], p_ref[h],
                        preferred_element_type=jnp.float32)
    srep = jnp.dot(s_ref[0] + s_ref[1], r_ref[...],
                   preferred_element_type=jnp.float32)
    return jnp.maximum(u / (srep + 1e-16), 0.0)


def _proj2_body(u_ref, s_ref, p_ref, r_ref, w_ref, b_ref, ms_ref, md_ref,
                xn_ref, as_ref, ad_ref):
    xi = _combine(u_ref, s_ref, p_ref, r_ref)
    xn = jnp.dot(xi, w_ref[...], preferred_element_type=jnp.float32)
    xn = xn + b_ref[...]
    xn_ref[...] = xn
    as_ref[...] = jnp.dot(xn, ms_ref[...], preferred_element_type=jnp.float32)
    ad_ref[...] = jnp.dot(xn, md_ref[...], preferred_element_type=jnp.float32)


_COMBINE_SPECS = [
    pl.BlockSpec((2, 8, BN2, 16), lambda i: (0, 0, i, 0)),
    pl.BlockSpec((2, BN2, 16), lambda i: (0, i, 0)),
    pl.BlockSpec((8, 16, HID), lambda i: (0, 0, 0)),
    pl.BlockSpec((16, HID), lambda i: (0, 0)),
]


def _proj2(u4, s32, w, b, ms, md):
    return pl.pallas_call(
        _proj2_body,
        grid=(N // BN2,),
        in_specs=_COMBINE_SPECS + [
            pl.BlockSpec((HID, HID), lambda i: (0, 0)),
            pl.BlockSpec((1, HID), lambda i: (0, 0)),
            pl.BlockSpec((HID, 16), lambda i: (0, 0)),
            pl.BlockSpec((HID, 16), lambda i: (0, 0)),
        ],
        out_specs=[
            pl.BlockSpec((BN2, HID), lambda i: (i, 0)),
            pl.BlockSpec((BN2, 16), lambda i: (i, 0)),
            pl.BlockSpec((BN2, 16), lambda i: (i, 0)),
        ],
        out_shape=[
            jax.ShapeDtypeStruct((N, HID), jnp.float32),
            jax.ShapeDtypeStruct((N, 16), jnp.float32),
            jax.ShapeDtypeStruct((N, 16), jnp.float32),
        ],
    )(u4, s32, _p_place(), _r_expand(), w, b.reshape(1, HID), ms, md)


def _final_body(u_ref, s_ref, p_ref, r_ref, w_ref, b_ref, o_ref):
    xi = _combine(u_ref, s_ref, p_ref, r_ref)
    o_ref[...] = jnp.dot(xi, w_ref[...],
                         preferred_element_type=jnp.float32) + b_ref[...]


def _final(u4, s32, w, b):
    return pl.pallas_call(
        _final_body,
        grid=(N // BN2,),
        in_specs=_COMBINE_SPECS + [
            pl.BlockSpec((HID, OUT), lambda i: (0, 0)),
            pl.BlockSpec((1, OUT), lambda i: (0, 0)),
        ],
        out_specs=pl.BlockSpec((BN2, OUT), lambda i: (i, 0)),
        out_shape=jax.ShapeDtypeStruct((N, OUT), jnp.float32),
    )(u4, s32, _p_place(), _r_expand(), w, b.reshape(1, OUT))


def _edges_sc(as0, ad0, xs0, sr0, ds0, as1, ad1, xs1, sr1, ds1):
    mesh = plsc.VectorSubcoreMesh(core_axis_name="c", subcore_axis_name="s")

    @functools.partial(
        pl.kernel,
        out_type=[
            jax.ShapeDtypeStruct((2, 2, NPAD, 16), jnp.float32),
            jax.ShapeDtypeStruct((2, 2, 8, NPAD, 16), jnp.float32),
            jax.ShapeDtypeStruct((2, EROWS, 128, 16), jnp.float32),
        ],
        mesh=mesh,
        scratch_types=[
            pltpu.VMEM((4, 128), jnp.int32),
            pltpu.VMEM((4, 128, 16), jnp.float32),
            pltpu.VMEM((4, 128, 16), jnp.float32),
            pltpu.VMEM((4, 128, 16), jnp.float32),
            pltpu.VMEM((160, 16), jnp.float32),
            pltpu.VMEM_SHARED((NPAD, 16), jnp.float32),
            pltpu.SemaphoreType.DMA,
            pltpu.SemaphoreType.DMA,
            pltpu.SemaphoreType.DMA,
        ],
        compiler_params=pltpu.CompilerParams(use_tc_tiling_on_sc=False),
    )
    def ek(as0r, ad0r, xs0r, sr0r, ds0r, as1r, ad1r, xs1r, sr1r, ds1r,
           s_out, un_out, ex_out,
           idx8, arows, brows, xrows, zz, agg,
           sem, sem2, sem3):
        cid = lax.axis_index("c")
        sid = lax.axis_index("s")
        rbase = sid * RS
        zvec = jnp.zeros((16,), jnp.float32)
        GRID = EPAD // W

        @pl.loop(0, 160)
        def _(i):
            zz[i, :] = zvec

        def zero_slice():
            for j in range(20):
                pltpu.sync_copy(zz, agg.at[pl.ds(rbase + j * 160, 160)])

        idx_specs = [pl.BlockSpec((4, 128), lambda w: (w, 0)),
                     pl.BlockSpec((4, 128), lambda w: (w, 0))]
        ex_spec = pl.BlockSpec((4, 128, 16), lambda w: (w, 0, 0))

        for et, (asr, adr, xsr, srr, dsr) in enumerate(
            ((as0r, ad0r, xs0r, sr0r, ds0r),
             (as1r, ad1r, xs1r, sr1r, ds1r))):
            zero_slice()
            plsc.subcore_barrier()

            def p1_body(idx_s_v, idx_d_v, ex_v, asr=asr, adr=adr):
                ga = [pltpu.async_copy(asr.at[idx_s_v.at[k]], arows.at[k],
                                       sem) for k in range(4)]
                gb = [pltpu.async_copy(adr.at[idx_d_v.at[k]], brows.at[k],
                                       sem2) for k in range(4)]
                scs = []
                for k in range(4):
                    ga[k].wait()
                    gb[k].wait()

                    @pl.loop(0, 128, step=2)
                    def _(r, k=k):
                        for rr in range(2):
                            t0 = arows[k, r + rr, :] + brows[k, r + rr, :]
                            t0 = jnp.where(t0 >= 0.0, t0, t0 * 0.2)
                            ex_v[k, r + rr, :] = jnp.exp(t0)

                    scs.append(pltpu.async_copy(
                        ex_v.at[k], agg.at[idx_d_v.at[k]], sem3, add=True))
                for cp in scs:
                    cp.wait()

            pltpu.emit_pipeline(
                p1_body,
                grid=(GRID,),
                in_specs=idx_specs,
                out_specs=[ex_spec],
                core_axis_name=("c", "s"),
                dimension_semantics=(pltpu.PARALLEL,),
            )(srr, dsr, ex_out.at[et])

            plsc.subcore_barrier()
            pltpu.sync_copy(agg.at[pl.ds(rbase, RS)],
                            s_out.at[et, cid, pl.ds(rbase, RS)])
            zero_slice()
            plsc.subcore_barrier()

            for h in range(8):
                def p2_body(idx_s_v, idx_d_v, ex_v, h=h, xsr=xsr):
                    for k in range(4):
                        for j in range(8):
                            sl = pl.ds(j * 16, 16)
                            idx8[k, sl] = idx_s_v[k, sl] * 8 + h
                    cps = [pltpu.async_copy(xsr.at[idx8.at[k]], xrows.at[k],
                                            sem) for k in range(4)]
                    scs = []
                    for k in range(4):
                        cps[k].wait()

                        @pl.loop(0, 128, step=2)
                        def _(r, k=k):
                            for rr in range(2):
                                ev = ex_v[k, r + rr, :]
                                xrows[k, r + rr, :] = (
                                    xrows[k, r + rr, :] * ev[h])

                        scs.append(pltpu.async_copy(
                            xrows.at[k], agg.at[idx_d_v.at[k]], sem3,
                            add=True))
                    for cp in scs:
                        cp.wait()

                pltpu.emit_pipeline(
                    p2_body,
                    grid=(GRID,),
                    in_specs=idx_specs + [ex_spec],
                    out_specs=[],
                    core_axis_name=("c", "s"),
                    dimension_semantics=(pltpu.PARALLEL,),
                )(srr, dsr, ex_out.at[et])

                plsc.subcore_barrier()
                pltpu.sync_copy(agg.at[pl.ds(rbase, RS)],
                                un_out.at[et, cid, h, pl.ds(rbase, RS)])
                zero_slice()
                plsc.subcore_barrier()

    return ek(as0, ad0, xs0, sr0, ds0, as1, ad1, xs1, sr1, ds1)


def _pad_idx(a, off):
    pad = jnp.arange(EPAD - E, dtype=jnp.int32)
    fill = off + (pad % 1024)
    return jnp.concatenate([a.astype(jnp.int32), fill]).reshape(EROWS, 128)


def kernel(x_author, x_paper, edge_index_writes, edge_index_rev_writes, params):
    p1, p2 = params["layers"]
    kw = "author__writes__paper"
    kr = "paper__rev_writes__author"

    sw = _pad_idx(edge_index_writes[0], 0)
    dw = _pad_idx(edge_index_writes[1], N)
    sr = _pad_idx(edge_index_rev_writes[0], 0)
    dr = _pad_idx(edge_index_rev_writes[1], N)

    def layer(proj_a, proj_p, p):
        xn_a, as_a, ad_a = proj_a
        xn_p, as_p, ad_p = proj_p
        s_out, un_out, _ = _edges_sc(
            as_a, ad_p, xn_a.reshape(8 * N, 16), sw, dw,
            as_p, ad_a, xn_p.reshape(8 * N, 16), sr, dr,
        )
        return (un_out[1], s_out[1]), (un_out[0], s_out[0])

    def mk_ms(p):
        return (_mk_m16(p["att_src"][kw]), _mk_m16(p["att_dst"][kr]),
                _mk_m16(p["att_src"][kr]), _mk_m16(p["att_dst"][kw]))

    ms_a, md_a, ms_p, md_p = mk_ms(p1)
    proj_a = _proj1(x_author, p1["proj"]["author"]["W"],
                    p1["proj"]["author"]["b"], ms_a, md_a)
    proj_p = _proj1(x_paper, p1["proj"]["paper"]["W"],
                    p1["proj"]["paper"]["b"], ms_p, md_p)
    (u_a, s_a), (u_p, s_p) = layer(proj_a, proj_p, p1)

    ms_a, md_a, ms_p, md_p = mk_ms(p2)
    proj_a2 = _proj2(u_a, s_a, p2["proj"]["author"]["W"],
                     p2["proj"]["author"]["b"], ms_a, md_a)
    proj_p2 = _proj2(u_p, s_p, p2["proj"]["paper"]["W"],
                     p2["proj"]["paper"]["b"], ms_p, md_p)
    (u_a2, s_a2), (u_p2, s_p2) = layer(proj_a2, proj_p2, p2)

    return _final(u_p2, s_p2, params["lin_W"], params["lin_b"])
